# baseline (speedup 1.0000x reference)
import functools

import jax
import jax.numpy as jnp
from jax import lax
from jax.experimental import pallas as pl
from jax.experimental.pallas import tpu as pltpu
from jax.experimental.pallas import tpu_sc as plsc

N = 100000
D = 128
NW = 32
PW = N // NW
CW = 480
NFULL = PW // CW
LAST = PW - NFULL * CW
NT = NFULL + 1

_mesh = plsc.VectorSubcoreMesh(core_axis_name="c", subcore_axis_name="s")


@functools.partial(
    pl.kernel,
    mesh=_mesh,
    out_type=jax.ShapeDtypeStruct((NW, PW, D), jnp.float32),
    scratch_types=[
        pltpu.VMEM((PW,), jnp.int32),
        pltpu.VMEM((CW, D), jnp.float32),
        pltpu.VMEM((CW, D), jnp.float32),
        pltpu.VMEM_SHARED((10, D), jnp.float32),
        pltpu.SemaphoreType.DMA,
        pltpu.SemaphoreType.DMA,
    ],
)
def _embed_gather(idx_hbm, table_hbm, out_hbm, idx_v, rows0, rows1,
                  table_sh, sem_g, sem_s):
    wid = lax.axis_index("s") * 2 + lax.axis_index("c")
    rows_bufs = (rows0, rows1)

    @pl.when(lax.axis_index("s") == 0)
    def _():
        pltpu.sync_copy(table_hbm, table_sh)

    pltpu.sync_copy(idx_hbm.at[wid], idx_v)
    plsc.subcore_barrier()

    scat_h = [None] * NT

    def do_round(t, size):
        b = t % 2
        if t >= 2:
            scat_h[t - 2].wait()
        idx_sl = idx_v.at[pl.ds(t * CW, size)]
        rows_sl = rows_bufs[b] if size == CW else rows_bufs[b].at[pl.ds(0, size)]
        pltpu.async_copy(table_sh.at[idx_sl], rows_sl, sem_g).wait()
        scat_h[t] = pltpu.async_copy(
            rows_sl, out_hbm.at[wid, pl.ds(t * CW, size)], sem_s)

    for t in range(NFULL):
        do_round(t, CW)
    do_round(NFULL, LAST)

    scat_h[NT - 2].wait()
    scat_h[NT - 1].wait()


def kernel(centrality, embedding):
    idx = centrality.astype(jnp.int32).reshape(NW, PW)
    return _embed_gather(idx, embedding).reshape(N, D)


# speedup vs baseline: 1.6574x; 1.6574x over previous
import functools

import jax
import jax.numpy as jnp
from jax import lax
from jax.experimental import pallas as pl
from jax.experimental.pallas import tpu as pltpu
from jax.experimental.pallas import tpu_sc as plsc

N = 100000
D = 128
NW = 32
CHUNK = 256
NCH = N // CHUNK
FULL_T = NCH // NW
REM = NCH - FULL_T * NW
TAIL = N - NCH * CHUNK
TAIL_BASE = NCH * CHUNK
TAIL_WID = NW - 1

_mesh = plsc.VectorSubcoreMesh(core_axis_name="c", subcore_axis_name="s")


@functools.partial(
    pl.kernel,
    mesh=_mesh,
    out_type=jax.ShapeDtypeStruct((N, D), jnp.float32),
    scratch_types=[
        pltpu.VMEM((CHUNK,), jnp.int32),
        pltpu.VMEM((CHUNK,), jnp.int32),
        pltpu.VMEM((CHUNK, D), jnp.float32),
        pltpu.VMEM((CHUNK, D), jnp.float32),
        pltpu.VMEM((TAIL,), jnp.int32),
        pltpu.VMEM((TAIL, D), jnp.float32),
        pltpu.VMEM_SHARED((10, D), jnp.float32),
        pltpu.SemaphoreType.DMA,
        pltpu.SemaphoreType.DMA,
    ],
)
def _embed_gather(idx_hbm, table_hbm, out_hbm, idx0, idx1, rows0, rows1,
                  idx_t, rows_t, table_sh, sem_g, sem_s):
    wid = lax.axis_index("s") * 2 + lax.axis_index("c")
    idx_bufs = (idx0, idx1)
    rows_bufs = (rows0, rows1)

    @pl.when(lax.axis_index("s") == 0)
    def _():
        pltpu.sync_copy(table_hbm, table_sh)

    def base(t):
        return (wid + t * NW) * CHUNK

    def load_idx(t):
        pltpu.sync_copy(idx_hbm.at[pl.ds(base(t), CHUNK)], idx_bufs[t % 2])

    scat_h = [None] * (FULL_T + 1)

    load_idx(0)
    plsc.subcore_barrier()

    def do_round(t):
        b = t % 2
        if t >= 2:
            scat_h[t - 2].wait()
        pltpu.async_copy(table_sh.at[idx_bufs[b]], rows_bufs[b], sem_g).wait()
        scat_h[t] = pltpu.async_copy(
            rows_bufs[b], out_hbm.at[pl.ds(base(t), CHUNK)], sem_s)
        if t + 1 < FULL_T:
            load_idx(t + 1)
        elif t + 1 == FULL_T:
            @pl.when(wid < REM)
            def _():
                load_idx(FULL_T)

    for t in range(FULL_T):
        do_round(t)

    @pl.when(wid < REM)
    def _():
        b = FULL_T % 2
        scat_h[FULL_T - 2].wait()
        pltpu.async_copy(table_sh.at[idx_bufs[b]], rows_bufs[b], sem_g).wait()
        pltpu.async_copy(rows_bufs[b], out_hbm.at[pl.ds(base(FULL_T), CHUNK)],
                         sem_s)

    @pl.when(wid == TAIL_WID)
    def _():
        pltpu.sync_copy(idx_hbm.at[pl.ds(TAIL_BASE, TAIL)], idx_t)
        pltpu.async_copy(table_sh.at[idx_t], rows_t, sem_g).wait()
        pltpu.async_copy(rows_t, out_hbm.at[pl.ds(TAIL_BASE, TAIL)],
                         sem_s).wait()

    scat_h[FULL_T - 2].wait()
    scat_h[FULL_T - 1].wait()


def kernel(centrality, embedding):
    idx = centrality.astype(jnp.int32)
    return _embed_gather(idx, embedding)


# speedup vs baseline: 1.8932x; 1.1423x over previous
import functools

import jax
import jax.numpy as jnp
from jax import lax
from jax.experimental import pallas as pl
from jax.experimental.pallas import tpu as pltpu
from jax.experimental.pallas import tpu_sc as plsc

N = 100000
D = 128
NW = 32
CHUNK = 256
NCH = N // CHUNK
FULL_T = NCH // NW
REM = NCH - FULL_T * NW
TAIL = N - NCH * CHUNK
TAIL_BASE = NCH * CHUNK
TAIL_WID = NW - 1
NBUF = 3

_mesh = plsc.VectorSubcoreMesh(core_axis_name="c", subcore_axis_name="s")


@functools.partial(
    pl.kernel,
    mesh=_mesh,
    out_type=jax.ShapeDtypeStruct((N, D), jnp.float32),
    scratch_types=[
        pltpu.VMEM((CHUNK,), jnp.int32),
        pltpu.VMEM((CHUNK,), jnp.int32),
        pltpu.VMEM((CHUNK,), jnp.int32),
        pltpu.VMEM((CHUNK, D), jnp.float32),
        pltpu.VMEM((CHUNK, D), jnp.float32),
        pltpu.VMEM((CHUNK, D), jnp.float32),
        pltpu.VMEM((TAIL,), jnp.int32),
        pltpu.VMEM((TAIL, D), jnp.float32),
        pltpu.VMEM_SHARED((10, D), jnp.float32),
        pltpu.SemaphoreType.DMA,
        pltpu.SemaphoreType.DMA,
    ],
)
def _embed_gather(idx_hbm, table_hbm, out_hbm, idx0, idx1, idx2,
                  rows0, rows1, rows2, idx_t, rows_t, table_sh,
                  sem_g, sem_s):
    wid = lax.axis_index("s") * 2 + lax.axis_index("c")
    idx_bufs = (idx0, idx1, idx2)
    rows_bufs = (rows0, rows1, rows2)

    @pl.when(lax.axis_index("s") == 0)
    def _():
        pltpu.sync_copy(table_hbm, table_sh)

    def base(t):
        return (wid + t * NW) * CHUNK

    def load_idx(t):
        pltpu.sync_copy(idx_hbm.at[pl.ds(base(t), CHUNK)], idx_bufs[t % NBUF])

    def start_gather(t):
        return pltpu.async_copy(table_sh.at[idx_bufs[t % NBUF]],
                                rows_bufs[t % NBUF], sem_g)

    g_h = [None] * (FULL_T + 1)
    scat_h = [None] * (FULL_T + 1)

    load_idx(0)
    load_idx(1)
    plsc.subcore_barrier()
    g_h[0] = start_gather(0)

    for t in range(FULL_T):
        if t >= 2:
            scat_h[t - 2].wait()
        if t + 1 < FULL_T:
            g_h[t + 1] = start_gather(t + 1)
        elif t + 1 == FULL_T:
            @pl.when(wid < REM)
            def _():
                g_h[FULL_T] = start_gather(FULL_T)
        g_h[t].wait()
        scat_h[t] = pltpu.async_copy(
            rows_bufs[t % NBUF], out_hbm.at[pl.ds(base(t), CHUNK)], sem_s)
        if t + 2 < FULL_T:
            load_idx(t + 2)
        elif t + 2 == FULL_T:
            @pl.when(wid < REM)
            def _():
                load_idx(FULL_T)

    @pl.when(wid < REM)
    def _():
        g_h[FULL_T].wait()
        pltpu.async_copy(rows_bufs[FULL_T % NBUF],
                         out_hbm.at[pl.ds(base(FULL_T), CHUNK)], sem_s)
        scat_h[FULL_T - 2].wait()

    @pl.when(wid == TAIL_WID)
    def _():
        pltpu.sync_copy(idx_hbm.at[pl.ds(TAIL_BASE, TAIL)], idx_t)
        pltpu.async_copy(table_sh.at[idx_t], rows_t, sem_g).wait()
        pltpu.async_copy(rows_t, out_hbm.at[pl.ds(TAIL_BASE, TAIL)],
                         sem_s).wait()

    scat_h[FULL_T - 2].wait()
    scat_h[FULL_T - 1].wait()


def kernel(centrality, embedding):
    idx = centrality.astype(jnp.int32)
    return _embed_gather(idx, embedding)
